# Initial kernel scaffold; baseline (speedup 1.0000x reference)
import jax
import jax.numpy as jnp
from jax.experimental import pallas as pl


def kernel(x, gate_weight, W1, b1, W2, b2):
    raise NotImplementedError("write your pallas kernel here")


# speedup vs baseline: 1.2410x; 1.2410x over previous
import functools

import numpy as np
import jax
import jax.numpy as jnp
from jax import lax
from jax.experimental import pallas as pl
from jax.experimental.pallas import tpu as pltpu

E = 8
D = 768
F = 3072
S = 2048
K = 2 * S

_SQRT_2_OVER_PI = np.sqrt(2.0 / np.pi).astype(np.float32)


def _gelu_tanh(v):
    return 0.5 * v * (1.0 + jnp.tanh(_SQRT_2_OVER_PI * (v + 0.044715 * v ** 3)))


def _router_body(scores_ref, g_ref):
    s = scores_ref[...]

    def bs_body(_, carry):
        lo, hi = carry
        mid = (lo + hi) // 2
        t = lax.bitcast_convert_type(mid, jnp.float32)
        cnt = jnp.sum((s >= t).astype(jnp.float32))
        big = cnt >= K
        return (jnp.where(big, mid, lo), jnp.where(big, hi, mid))

    lo0 = jnp.int32(0)
    hi0 = jnp.int32(0x40000000)
    lo, _ = lax.fori_loop(0, 31, bs_body, (lo0, hi0))
    thr = lax.bitcast_convert_type(lo, jnp.float32)

    gt = s > thr
    cnt_gt = jnp.sum(gt.astype(jnp.float32))
    need_eq = jnp.float32(K) - cnt_gt

    eq = (s == thr).astype(jnp.float32)
    t_idx = lax.broadcasted_iota(jnp.int32, (S, S), 0)
    s_idx = lax.broadcasted_iota(jnp.int32, (S, S), 1)
    strict_lt = (t_idx < s_idx).astype(jnp.float32)
    eqx = jnp.dot(eq, strict_lt, preferred_element_type=jnp.float32)
    rowtot = jnp.sum(eq, axis=1, keepdims=True)
    a0 = lax.broadcasted_iota(jnp.int32, (E, E), 0)
    a1 = lax.broadcasted_iota(jnp.int32, (E, E), 1)
    strict8 = (a0 > a1).astype(jnp.float32)
    offs = jnp.dot(strict8, rowtot, preferred_element_type=jnp.float32)
    erank = eqx + offs

    sel = gt | ((s == thr) & (erank < need_eq))
    g_ref[...] = jnp.where(sel, s, jnp.float32(0.0))


def _router(scoresT, interpret=False):
    return pl.pallas_call(
        _router_body,
        out_shape=jax.ShapeDtypeStruct((E, S), jnp.float32),
        interpret=interpret,
    )(scoresT)


_FB = 4
_FC = F // _FB


def _dense_body(x_ref, g_ref, w1_ref, b1_ref, w2_ref, b2_ref, out_ref):
    e = pl.program_id(0)
    fb = pl.program_id(1)

    xb = x_ref[...]
    h = jnp.dot(xb, w1_ref[0], preferred_element_type=jnp.float32)
    h = _gelu_tanh(h + b1_ref[0, 0, :][None, :])
    part = jnp.dot(h, w2_ref[0], preferred_element_type=jnp.float32)

    onehot = (lax.broadcasted_iota(jnp.int32, (E, 1), 0) == e).astype(jnp.float32)
    gvec = jnp.sum(g_ref[...] * onehot, axis=0)[:, None]
    contrib = part * gvec

    @pl.when(fb == 0)
    def _():
        contrib2 = contrib + b2_ref[0, 0, :][None, :] * gvec

        @pl.when(e == 0)
        def _():
            out_ref[...] = contrib2

        @pl.when(e != 0)
        def _():
            out_ref[...] += contrib2

    @pl.when(fb != 0)
    def _():
        out_ref[...] += contrib


def _dense(xf, G, W1, b1, W2, b2, interpret=False):
    b1r = b1.reshape(E, _FB, _FC).reshape(E * _FB, 1, _FC)
    b2r = b2.reshape(E, 1, D)
    grid = (E, _FB)
    return pl.pallas_call(
        _dense_body,
        grid=grid,
        in_specs=[
            pl.BlockSpec((S, D), lambda e, fb: (0, 0)),
            pl.BlockSpec((E, S), lambda e, fb: (0, 0)),
            pl.BlockSpec((1, D, _FC), lambda e, fb: (e, 0, fb)),
            pl.BlockSpec((1, 1, _FC), lambda e, fb: (e * _FB + fb, 0, 0)),
            pl.BlockSpec((1, _FC, D), lambda e, fb: (e, fb, 0)),
            pl.BlockSpec((1, 1, D), lambda e, fb: (e, 0, 0)),
        ],
        out_specs=pl.BlockSpec((S, D), lambda e, fb: (0, 0)),
        out_shape=jax.ShapeDtypeStruct((S, D), jnp.float32),
        compiler_params=pltpu.CompilerParams(
            dimension_semantics=("arbitrary", "arbitrary"),
        ),
        interpret=interpret,
    )(xf, G, W1, b1r, W2, b2r)


def kernel(x, gate_weight, W1, b1, W2, b2):
    Bc, s_len, Dc = x.shape
    xf = x.reshape(-1, Dc)
    logits = xf @ gate_weight.T
    scoresT = jax.nn.softmax(logits, axis=-1).T
    G = _router(scoresT)
    y = _dense(xf, G, W1, b1, W2, b2)
    return y.reshape(Bc, s_len, Dc)
